# baseline (speedup 1.0000x reference)
import functools
import math

import jax
import jax.numpy as jnp
from jax import lax
from jax.experimental import pallas as pl
from jax.experimental.pallas import tpu as pltpu
from jax.experimental.pallas import tpu_sc as plsc

EMBED_DIM = 32
SCALE = math.sqrt(EMBED_DIM)

NUM_CORES = 2
NUM_SUBCORES = 16
NUM_WORKERS = NUM_CORES * NUM_SUBCORES

CHUNK = 128
PAIR = 2
NBUF = 4
GDEPTH = 3
LANES = 16


def _make_lookup(n_idx: int):
    group = PAIR * CHUNK
    assert n_idx % (NUM_WORKERS * group) == 0
    per_w = n_idx // NUM_WORKERS
    nch = per_w // CHUNK
    ngrp = per_w // group
    assert ngrp % NBUF == 0 and ngrp >= 2 * NBUF

    mesh = plsc.VectorSubcoreMesh(
        core_axis_name="c", subcore_axis_name="s",
        num_cores=NUM_CORES, num_subcores=NUM_SUBCORES)

    @functools.partial(
        pl.kernel,
        out_type=jax.ShapeDtypeStruct((n_idx // group, group, EMBED_DIM),
                                      jnp.float32),
        mesh=mesh,
        scratch_types=[
            pltpu.VMEM((per_w,), jnp.int32),
            pltpu.VMEM((NBUF, group, EMBED_DIM), jnp.bfloat16),
            pltpu.VMEM((NBUF, group, EMBED_DIM), jnp.float32),
        ] + [pltpu.SemaphoreType.DMA] * (2 * NBUF),
        compiler_params=pltpu.CompilerParams(use_tc_tiling_on_sc=False,
                                             needs_layout_passes=False),
    )
    def lookup(x_hbm, lut_hbm, out_hbm, idx_v, rows_v, fout_v, *sems):
        gsem = sems[:NBUF]
        wsem = sems[NBUF:]
        wid = lax.axis_index("s") * NUM_CORES + lax.axis_index("c")
        grp0 = wid * ngrp

        pltpu.sync_copy(x_hbm.at[pl.ds(wid * per_w, per_w)], idx_v)

        def idx_slice(g):
            return idx_v.at[pl.ds(lax.mul(g, group), group)]

        def issue_gather(buf, g):
            pltpu.async_copy(lut_hbm.at[idx_slice(g)], rows_v.at[buf],
                             gsem[buf])

        def wait_gather(buf, g):
            pltpu.make_async_copy(lut_hbm.at[idx_slice(g)], rows_v.at[buf],
                                  gsem[buf]).wait()

        def scale(buf):
            def body(i, _):
                v = rows_v[buf, i, pl.ds(0, EMBED_DIM)]
                a, b2 = plsc.unpack(v, format=plsc.PackFormat.INTERLEAVED,
                                    preferred_element_type=jnp.float32)
                fout_v[buf, i, pl.ds(0, LANES)] = a * SCALE
                fout_v[buf, i, pl.ds(LANES, LANES)] = b2 * SCALE
                return 0
            lax.fori_loop(0, group, body, 0, unroll=8)

        def issue_write(buf, g):
            pltpu.async_copy(fout_v.at[buf], out_hbm.at[grp0 + g], wsem[buf])

        def wait_write(buf, g):
            pltpu.make_async_copy(fout_v.at[buf], out_hbm.at[grp0 + g],
                                  wsem[buf]).wait()

        def step(g, b, first_block, last_block):
            wait_gather(b, g)
            scale(b)
            issue_write(b, g)
            if not (first_block and b < NBUF - GDEPTH):
                pb = (b + GDEPTH) % NBUF
                wait_write(pb, g + GDEPTH - NBUF)
            if not (last_block and b >= NBUF - GDEPTH):
                issue_gather((b + GDEPTH) % NBUF, g + GDEPTH)

        for b in range(GDEPTH):
            issue_gather(b, b)

        for b in range(NBUF):
            step(b, b, True, False)

        def outer(o, _):
            for b in range(NBUF):
                step(o * NBUF + b, b, False, False)
            return 0
        lax.fori_loop(1, ngrp // NBUF - 1, outer, 0)

        for b in range(NBUF):
            step(ngrp - NBUF + b, b, False, True)

        for g in range(ngrp - (NBUF - GDEPTH), ngrp):
            wait_write(g % NBUF, g)

    return lookup


def kernel(x, lut):
    b, s = x.shape
    n_idx = b * s
    x_flat = x.reshape(n_idx).astype(jnp.int32)
    v = lut.shape[0]
    lut_bf = (lut.reshape(v, 2, LANES).transpose(0, 2, 1)
              .reshape(v, EMBED_DIM).astype(jnp.bfloat16))
    out = _make_lookup(n_idx)(x_flat, lut_bf)
    return out.reshape(b, s, EMBED_DIM)


# speedup vs baseline: 1.2360x; 1.2360x over previous
import functools
import math

import jax
import jax.numpy as jnp
from jax import lax
from jax.experimental import pallas as pl
from jax.experimental.pallas import tpu as pltpu
from jax.experimental.pallas import tpu_sc as plsc

EMBED_DIM = 32
SCALE = math.sqrt(EMBED_DIM)

NUM_CORES = 2
NUM_SUBCORES = 16
NUM_WORKERS = NUM_CORES * NUM_SUBCORES

CHUNK = 128
PAIR = 2
NBUF = 4
GDEPTH = 3
LANES = 16


def _make_lookup(n_idx: int):
    group = PAIR * CHUNK
    assert n_idx % (NUM_WORKERS * group) == 0
    per_w = n_idx // NUM_WORKERS
    nch = per_w // CHUNK
    ngrp = per_w // group
    assert ngrp % NBUF == 0 and ngrp >= 2 * NBUF

    mesh = plsc.VectorSubcoreMesh(
        core_axis_name="c", subcore_axis_name="s",
        num_cores=NUM_CORES, num_subcores=NUM_SUBCORES)

    @functools.partial(
        pl.kernel,
        out_type=jax.ShapeDtypeStruct((n_idx // group, group, EMBED_DIM),
                                      jnp.float32),
        mesh=mesh,
        scratch_types=[
            pltpu.VMEM((per_w,), jnp.int32),
            pltpu.VMEM((NBUF, group, EMBED_DIM), jnp.bfloat16),
            pltpu.VMEM((NBUF, group, EMBED_DIM), jnp.float32),
        ] + [pltpu.SemaphoreType.DMA] * (2 * NBUF),
        compiler_params=pltpu.CompilerParams(use_tc_tiling_on_sc=False,
                                             needs_layout_passes=False),
    )
    def lookup(x_hbm, lut_hbm, out_hbm, idx_v, rows_v, fout_v, *sems):
        gsem = sems[:NBUF]
        wsem = sems[NBUF:]
        wid = lax.axis_index("s") * NUM_CORES + lax.axis_index("c")
        grp0 = wid * ngrp

        pltpu.sync_copy(x_hbm.at[pl.ds(wid * per_w, per_w)], idx_v)

        def idx_slice(g):
            return idx_v.at[pl.ds(lax.mul(g, group), group)]

        def issue_gather(buf, g):
            pltpu.async_copy(lut_hbm.at[idx_slice(g)], rows_v.at[buf],
                             gsem[buf])

        def wait_gather(buf, g):
            pltpu.make_async_copy(lut_hbm.at[idx_slice(g)], rows_v.at[buf],
                                  gsem[buf]).wait()

        def scale(buf):
            def body(i, _):
                v = rows_v[buf, i, pl.ds(0, EMBED_DIM)]
                a, b2 = plsc.unpack(v, format=plsc.PackFormat.INTERLEAVED,
                                    preferred_element_type=jnp.float32)
                fout_v[buf, i, pl.ds(0, LANES)] = a * SCALE
                fout_v[buf, i, pl.ds(LANES, LANES)] = b2 * SCALE
                return 0
            lax.fori_loop(0, group, body, 0, unroll=8)

        def issue_write(buf, g):
            pltpu.async_copy(fout_v.at[buf], out_hbm.at[grp0 + g], wsem[buf])

        def wait_write(buf, g):
            pltpu.make_async_copy(fout_v.at[buf], out_hbm.at[grp0 + g],
                                  wsem[buf]).wait()

        def step(g, b, first_block, last_block):
            wait_gather(b, g)
            scale(b)
            issue_write(b, g)
            if not (first_block and b < NBUF - GDEPTH):
                pb = (b + GDEPTH) % NBUF
                wait_write(pb, g + GDEPTH - NBUF)
            if not (last_block and b >= NBUF - GDEPTH):
                issue_gather((b + GDEPTH) % NBUF, g + GDEPTH)

        for b in range(GDEPTH):
            issue_gather(b, b)

        for b in range(NBUF):
            step(b, b, True, False)

        def outer(o, _):
            for b in range(NBUF):
                step(o * NBUF + b, b, False, False)
            return 0
        lax.fori_loop(1, ngrp // NBUF - 1, outer, 0)

        for b in range(NBUF):
            step(ngrp - NBUF + b, b, False, True)

        for g in range(ngrp - (NBUF - GDEPTH), ngrp):
            wait_write(g % NBUF, g)

    return lookup


def kernel(x, lut):
    b, s = x.shape
    n_idx = b * s
    x_flat = x.reshape(n_idx).astype(jnp.int32)
    lut_bf = lut.astype(jnp.bfloat16)
    out = _make_lookup(n_idx)(x_flat, lut_bf)
    return out.reshape(b, s, EMBED_DIM)


# speedup vs baseline: 1.4553x; 1.1775x over previous
import functools
import math

import jax
import jax.numpy as jnp
from jax import lax
from jax.experimental import pallas as pl
from jax.experimental.pallas import tpu as pltpu
from jax.experimental.pallas import tpu_sc as plsc

EMBED_DIM = 32
SCALE = math.sqrt(EMBED_DIM)

NUM_CORES = 2
NUM_SUBCORES = 16
NUM_WORKERS = NUM_CORES * NUM_SUBCORES

CHUNK = 128
PAIR = 2
NBUF = 4
GDEPTH = 3
LANES = 16


def _make_lookup(n_idx: int):
    group = PAIR * CHUNK
    assert n_idx % (NUM_WORKERS * group) == 0
    per_w = n_idx // NUM_WORKERS
    nch = per_w // CHUNK
    ngrp = per_w // group
    assert ngrp % NBUF == 0 and ngrp >= 2 * NBUF

    mesh = plsc.VectorSubcoreMesh(
        core_axis_name="c", subcore_axis_name="s",
        num_cores=NUM_CORES, num_subcores=NUM_SUBCORES)

    @functools.partial(
        pl.kernel,
        out_type=jax.ShapeDtypeStruct((n_idx // group, group, EMBED_DIM),
                                      jnp.float32),
        mesh=mesh,
        scratch_types=[
            pltpu.VMEM((per_w,), jnp.int32),
            pltpu.VMEM((NBUF, group, EMBED_DIM), jnp.float32),
        ] + [pltpu.SemaphoreType.DMA] * (2 * NBUF),
        compiler_params=pltpu.CompilerParams(use_tc_tiling_on_sc=False,
                                             needs_layout_passes=False),
    )
    def lookup(x_hbm, lut_hbm, out_hbm, idx_v, rows_v, *sems):
        gsem = sems[:NBUF]
        wsem = sems[NBUF:]
        wid = lax.axis_index("s") * NUM_CORES + lax.axis_index("c")
        grp0 = wid * ngrp

        pltpu.sync_copy(x_hbm.at[pl.ds(wid * per_w, per_w)], idx_v)

        def idx_slice(g):
            return idx_v.at[pl.ds(lax.mul(g, group), group)]

        def issue_gather(buf, g):
            pltpu.async_copy(lut_hbm.at[idx_slice(g)], rows_v.at[buf],
                             gsem[buf])

        def wait_gather(buf, g):
            pltpu.make_async_copy(lut_hbm.at[idx_slice(g)], rows_v.at[buf],
                                  gsem[buf]).wait()

        def scale(buf):
            def body(i, _):
                for j in range(EMBED_DIM // LANES):
                    sl = pl.ds(j * LANES, LANES)
                    rows_v[buf, i, sl] = rows_v[buf, i, sl] * SCALE
                return 0
            lax.fori_loop(0, group, body, 0, unroll=8)

        def issue_write(buf, g):
            pltpu.async_copy(rows_v.at[buf], out_hbm.at[grp0 + g], wsem[buf])

        def wait_write(buf, g):
            pltpu.make_async_copy(rows_v.at[buf], out_hbm.at[grp0 + g],
                                  wsem[buf]).wait()

        def step(g, b, first_block, last_block):
            wait_gather(b, g)
            scale(b)
            issue_write(b, g)
            if not (first_block and b < NBUF - GDEPTH):
                pb = (b + GDEPTH) % NBUF
                wait_write(pb, g + GDEPTH - NBUF)
            if not (last_block and b >= NBUF - GDEPTH):
                issue_gather((b + GDEPTH) % NBUF, g + GDEPTH)

        for b in range(GDEPTH):
            issue_gather(b, b)

        for b in range(NBUF):
            step(b, b, True, False)

        def outer(o, _):
            for b in range(NBUF):
                step(o * NBUF + b, b, False, False)
            return 0
        lax.fori_loop(1, ngrp // NBUF - 1, outer, 0)

        for b in range(NBUF):
            step(ngrp - NBUF + b, b, False, True)

        for g in range(ngrp - (NBUF - GDEPTH), ngrp):
            wait_write(g % NBUF, g)

    return lookup


def kernel(x, lut):
    b, s = x.shape
    n_idx = b * s
    x_flat = x.reshape(n_idx).astype(jnp.int32)
    out = _make_lookup(n_idx)(x_flat, lut)
    return out.reshape(b, s, EMBED_DIM)
